# baseline (device time: 69989 ns/iter reference)
import jax
import jax.numpy as jnp
from jax import lax
from jax.experimental import pallas as pl
from jax.experimental.pallas import tpu as pltpu


def kernel(dy, W):
    m, f = dy.shape
    d = W.shape[0]
    half = m // 2

    def body(dy_ref, w_ref, out_ref, acc_ref, rx_ref, ry_ref,
             send_x, recv_x, send_y, recv_y):
        my_x = lax.axis_index("x")
        my_y = lax.axis_index("y")
        x_nbr = (1 - my_x, my_y)
        y_nbr = (my_x, 1 - my_y)

        barrier_sem = pltpu.get_barrier_semaphore()
        for nbr in (x_nbr, y_nbr):
            pl.semaphore_signal(
                barrier_sem, inc=1,
                device_id=nbr, device_id_type=pl.DeviceIdType.MESH,
            )
        pl.semaphore_wait(barrier_sem, 2)

        row0 = my_y * half
        acc_ref[...] = lax.dot_general(
            dy_ref[pl.ds(row0, half), :],
            w_ref[...],
            dimension_numbers=(((1,), (1,)), ((), ())),
            preferred_element_type=jnp.float32,
        )

        rdma_x = pltpu.make_async_remote_copy(
            src_ref=acc_ref,
            dst_ref=rx_ref,
            send_sem=send_x,
            recv_sem=recv_x,
            device_id=x_nbr,
            device_id_type=pl.DeviceIdType.MESH,
        )
        rdma_x.start()
        rdma_x.wait()
        acc_ref[...] = acc_ref[...] + rx_ref[...]
        out_ref[pl.ds(row0, half), :] = acc_ref[...]

        rdma_y = pltpu.make_async_remote_copy(
            src_ref=acc_ref,
            dst_ref=ry_ref,
            send_sem=send_y,
            recv_sem=recv_y,
            device_id=y_nbr,
            device_id_type=pl.DeviceIdType.MESH,
        )
        rdma_y.start()
        rdma_y.wait()
        out_ref[pl.ds((1 - my_y) * half, half), :] = ry_ref[...]

    return pl.pallas_call(
        body,
        out_shape=jax.ShapeDtypeStruct((m, d), jnp.float32),
        in_specs=[
            pl.BlockSpec(memory_space=pltpu.VMEM),
            pl.BlockSpec(memory_space=pltpu.VMEM),
        ],
        out_specs=pl.BlockSpec(memory_space=pltpu.VMEM),
        scratch_shapes=[
            pltpu.VMEM((half, d), jnp.float32),
            pltpu.VMEM((half, d), jnp.float32),
            pltpu.VMEM((half, d), jnp.float32),
            pltpu.SemaphoreType.DMA,
            pltpu.SemaphoreType.DMA,
            pltpu.SemaphoreType.DMA,
            pltpu.SemaphoreType.DMA,
        ],
        compiler_params=pltpu.CompilerParams(collective_id=0),
    )(dy, W)


# device time: 63322 ns/iter; 1.1053x vs baseline; 1.1053x over previous
import jax
import jax.numpy as jnp
from jax import lax
from jax.experimental import pallas as pl
from jax.experimental.pallas import tpu as pltpu

NCHUNK = 8


def kernel(dy, W):
    m, f = dy.shape
    d = W.shape[0]
    half = m // 2
    ch = half // NCHUNK

    def body(dy_ref, w_ref, out_ref, acc_ref, rx_ref, red_ref, ry_ref,
             send_x, recv_x, send_y, recv_y):
        my_x = lax.axis_index("x")
        my_y = lax.axis_index("y")
        x_nbr = (1 - my_x, my_y)
        y_nbr = (my_x, 1 - my_y)

        barrier_sem = pltpu.get_barrier_semaphore()
        for nbr in (x_nbr, y_nbr):
            pl.semaphore_signal(
                barrier_sem, inc=1,
                device_id=nbr, device_id_type=pl.DeviceIdType.MESH,
            )
        pl.semaphore_wait(barrier_sem, 2)

        row0 = my_y * half

        def rdma_x_chunk(c):
            return pltpu.make_async_remote_copy(
                src_ref=acc_ref.at[pl.ds(c * ch, ch)],
                dst_ref=rx_ref.at[pl.ds(c * ch, ch)],
                send_sem=send_x.at[c],
                recv_sem=recv_x.at[c],
                device_id=x_nbr,
                device_id_type=pl.DeviceIdType.MESH,
            )

        def rdma_y_chunk(c):
            return pltpu.make_async_remote_copy(
                src_ref=red_ref.at[pl.ds(c * ch, ch)],
                dst_ref=ry_ref.at[pl.ds(c * ch, ch)],
                send_sem=send_y.at[c],
                recv_sem=recv_y.at[c],
                device_id=y_nbr,
                device_id_type=pl.DeviceIdType.MESH,
            )

        for c in range(NCHUNK):
            acc_ref[pl.ds(c * ch, ch), :] = lax.dot_general(
                dy_ref[pl.ds(row0 + c * ch, ch), :],
                w_ref[...],
                dimension_numbers=(((1,), (1,)), ((), ())),
                preferred_element_type=jnp.float32,
            )
            rdma_x_chunk(c).start()

        for c in range(NCHUNK):
            r = rdma_x_chunk(c)
            r.wait_recv()
            red_ref[pl.ds(c * ch, ch), :] = (
                acc_ref[pl.ds(c * ch, ch), :] + rx_ref[pl.ds(c * ch, ch), :]
            )
            rdma_y_chunk(c).start()
            out_ref[pl.ds(row0 + c * ch, ch), :] = red_ref[pl.ds(c * ch, ch), :]
            r.wait_send()

        other0 = (1 - my_y) * half
        for c in range(NCHUNK):
            r = rdma_y_chunk(c)
            r.wait_recv()
            out_ref[pl.ds(other0 + c * ch, ch), :] = ry_ref[pl.ds(c * ch, ch), :]
            r.wait_send()

    return pl.pallas_call(
        body,
        out_shape=jax.ShapeDtypeStruct((m, d), jnp.float32),
        in_specs=[
            pl.BlockSpec(memory_space=pltpu.VMEM),
            pl.BlockSpec(memory_space=pltpu.VMEM),
        ],
        out_specs=pl.BlockSpec(memory_space=pltpu.VMEM),
        scratch_shapes=[
            pltpu.VMEM((half, d), jnp.float32),
            pltpu.VMEM((half, d), jnp.float32),
            pltpu.VMEM((half, d), jnp.float32),
            pltpu.VMEM((half, d), jnp.float32),
            pltpu.SemaphoreType.DMA((NCHUNK,)),
            pltpu.SemaphoreType.DMA((NCHUNK,)),
            pltpu.SemaphoreType.DMA((NCHUNK,)),
            pltpu.SemaphoreType.DMA((NCHUNK,)),
        ],
        compiler_params=pltpu.CompilerParams(collective_id=0),
    )(dy, W)


# device time: 50120 ns/iter; 1.3964x vs baseline; 1.2634x over previous
import os

import jax
import jax.numpy as jnp
from jax import lax
from jax.experimental import pallas as pl
from jax.experimental.pallas import tpu as pltpu

G = int(os.environ.get("KERNEL_G", "2"))
SUB = int(os.environ.get("KERNEL_SUB", "4"))
K = G * SUB


def kernel(dy, W):
    m, f = dy.shape
    d = W.shape[0]
    half = m // 2
    gch = half // G
    ch = half // K

    def body(dy_ref, w_ref, out_ref, acc_ref, rx_ref, red_ref, ry_ref,
             send_x, recv_x, send_y, recv_y):
        my_x = lax.axis_index("x")
        my_y = lax.axis_index("y")
        x_nbr = (1 - my_x, my_y)
        y_nbr = (my_x, 1 - my_y)

        barrier_sem = pltpu.get_barrier_semaphore()
        for nbr in (x_nbr, y_nbr):
            pl.semaphore_signal(
                barrier_sem, inc=1,
                device_id=nbr, device_id_type=pl.DeviceIdType.MESH,
            )
        pl.semaphore_wait(barrier_sem, 2)

        row0 = my_y * half

        def rdma_x_chunk(c):
            return pltpu.make_async_remote_copy(
                src_ref=acc_ref.at[pl.ds(c * ch, ch)],
                dst_ref=rx_ref.at[pl.ds(c * ch, ch)],
                send_sem=send_x.at[c],
                recv_sem=recv_x.at[c],
                device_id=x_nbr,
                device_id_type=pl.DeviceIdType.MESH,
            )

        def rdma_y_chunk(c):
            return pltpu.make_async_remote_copy(
                src_ref=red_ref.at[pl.ds(c * ch, ch)],
                dst_ref=ry_ref.at[pl.ds(c * ch, ch)],
                send_sem=send_y.at[c],
                recv_sem=recv_y.at[c],
                device_id=y_nbr,
                device_id_type=pl.DeviceIdType.MESH,
            )

        for g in range(G):
            acc_ref[pl.ds(g * gch, gch), :] = lax.dot_general(
                dy_ref[pl.ds(row0 + g * gch, gch), :],
                w_ref[...],
                dimension_numbers=(((1,), (1,)), ((), ())),
                preferred_element_type=jnp.float32,
            )
            for s in range(SUB):
                rdma_x_chunk(g * SUB + s).start()

        for c in range(K):
            r = rdma_x_chunk(c)
            r.wait_recv()
            red_ref[pl.ds(c * ch, ch), :] = (
                acc_ref[pl.ds(c * ch, ch), :] + rx_ref[pl.ds(c * ch, ch), :]
            )
            rdma_y_chunk(c).start()
            out_ref[pl.ds(row0 + c * ch, ch), :] = red_ref[pl.ds(c * ch, ch), :]
            r.wait_send()

        other0 = (1 - my_y) * half
        for c in range(K):
            r = rdma_y_chunk(c)
            r.wait_recv()
            out_ref[pl.ds(other0 + c * ch, ch), :] = ry_ref[pl.ds(c * ch, ch), :]
            r.wait_send()

    return pl.pallas_call(
        body,
        out_shape=jax.ShapeDtypeStruct((m, d), jnp.float32),
        in_specs=[
            pl.BlockSpec(memory_space=pltpu.VMEM),
            pl.BlockSpec(memory_space=pltpu.VMEM),
        ],
        out_specs=pl.BlockSpec(memory_space=pltpu.VMEM),
        scratch_shapes=[
            pltpu.VMEM((half, d), jnp.float32),
            pltpu.VMEM((half, d), jnp.float32),
            pltpu.VMEM((half, d), jnp.float32),
            pltpu.VMEM((half, d), jnp.float32),
            pltpu.SemaphoreType.DMA((K,)),
            pltpu.SemaphoreType.DMA((K,)),
            pltpu.SemaphoreType.DMA((K,)),
            pltpu.SemaphoreType.DMA((K,)),
        ],
        compiler_params=pltpu.CompilerParams(collective_id=0),
    )(dy, W)
